# baseline (speedup 1.0000x reference)
import jax
import jax.numpy as jnp
from jax.experimental import pallas as pl
from jax.experimental.pallas import tpu as pltpu

_B = 16
_H = 16
_S = 2048
_D = 64
_HB = 4


def _body(pos_ref, kc_ref, vc_ref, kval_ref, vval_ref, kout_ref, vout_ref):
    b = pl.program_id(0)
    r = pos_ref[b] - 1
    kout_ref[...] = kc_ref[...]
    vout_ref[...] = vc_ref[...]
    kout_ref[:, pl.ds(r, 1), :] = kval_ref[...]
    vout_ref[:, pl.ds(r, 1), :] = vval_ref[...]


def kernel(k_cache, v_cache, k_val, v_val, input_pos):
    cache_spec = pl.BlockSpec((None, _HB, _S, _D), lambda b, h, pos: (b, h, 0, 0))
    val_spec = pl.BlockSpec((None, _HB, 1, _D), lambda b, h, pos: (b, h, 0, 0))
    grid_spec = pltpu.PrefetchScalarGridSpec(
        num_scalar_prefetch=1,
        grid=(_B, _H // _HB),
        in_specs=[cache_spec, cache_spec, val_spec, val_spec],
        out_specs=[cache_spec, cache_spec],
    )
    out_shape = jax.ShapeDtypeStruct((_B, _H, _S, _D), jnp.float32)
    return pl.pallas_call(
        _body,
        grid_spec=grid_spec,
        out_shape=[out_shape, out_shape],
        compiler_params=pltpu.CompilerParams(
            dimension_semantics=("arbitrary", "arbitrary"),
            vmem_limit_bytes=100 * 1024 * 1024,
        ),
    )(input_pos, k_cache, v_cache, k_val, v_val)
